# baseline (speedup 1.0000x reference)
import functools

import jax
import jax.numpy as jnp
from jax import lax
from jax.experimental import pallas as pl
from jax.experimental.pallas import tpu as pltpu
from jax.experimental.pallas import tpu_sc as plsc

N_NODES = 100000
N_EDGES = 6400000
NVP = 100352
VPW = NVP // 32
FPW = VPW * 3
NW = 32
CH = 2048
NWIN = N_EDGES // CH
WIN_Q = NWIN // NW
WIN_R = NWIN % NW
PIPE = 96
W4 = 16384
NSTEP = PIPE * CH // W4
SLICE = NVP // 16


def _sc_scatter(dst, val):
    mesh = plsc.VectorSubcoreMesh(core_axis_name="c", subcore_axis_name="s")

    @functools.partial(
        pl.kernel,
        out_type=jax.ShapeDtypeStruct((2 * NVP,), jnp.float32),
        mesh=mesh,
        scratch_types=[
            pltpu.VMEM_SHARED((NVP,), jnp.float32),
            pltpu.VMEM((W4,), jnp.int32),
            pltpu.VMEM((W4,), jnp.float32),
            pltpu.VMEM((W4,), jnp.int32),
            pltpu.VMEM((W4,), jnp.float32),
            pltpu.VMEM((CH,), jnp.int32),
            pltpu.VMEM((CH,), jnp.float32),
            pltpu.VMEM((SLICE,), jnp.float32),
            pltpu.SemaphoreType.DMA,
            pltpu.SemaphoreType.DMA,
        ],
    )
    def k(dst_h, val_h, out_h, acc, idx_b0, val_b0, idx_b1, val_b1,
          idx_t, val_t, slice_b, sem0, sem1):
        c = lax.axis_index("c")
        s = lax.axis_index("s")
        w = c * 16 + s

        zero16 = jnp.zeros((16,), jnp.float32)

        def zero_body(i, carry):
            slice_b[pl.ds(i * 16, 16)] = zero16
            return carry

        lax.fori_loop(0, SLICE // 16, zero_body, 0)
        pltpu.sync_copy(slice_b, acc.at[pl.ds(s * SLICE, SLICE)])
        plsc.subcore_barrier()

        win0 = w * WIN_Q + jnp.minimum(w, WIN_R)
        nwin = WIN_Q + jnp.where(w < WIN_R, 1, 0)
        bufs = ((idx_b0, val_b0, sem0), (idx_b1, val_b1, sem1))

        def start_load(step, b):
            idx_b, val_b, sem = bufs[b]
            e = win0 * CH + step * W4
            pltpu.async_copy(dst_h.at[0, pl.ds(e, W4)], idx_b, sem)
            pltpu.async_copy(val_h.at[pl.ds(e, W4)], val_b, sem)

        def wait_load(b):
            idx_b, val_b, sem = bufs[b]
            pltpu.make_async_copy(dst_h.at[0, pl.ds(0, W4)], idx_b, sem).wait()
            pltpu.make_async_copy(val_h.at[pl.ds(0, W4)], val_b, sem).wait()

        def scatter(b):
            idx_b, val_b, _ = bufs[b]
            pltpu.sync_copy(val_b, acc.at[idx_b], add=True)

        start_load(0, 0)
        start_load(1, 1)

        def pair_body(j, carry):
            wait_load(0)
            scatter(0)
            start_load(2 * j + 2, 0)
            wait_load(1)
            scatter(1)
            start_load(2 * j + 3, 1)
            return carry

        lax.fori_loop(0, NSTEP // 2 - 1, pair_body, 0)
        wait_load(0)
        scatter(0)
        wait_load(1)
        scatter(1)

        def tail_body(t, carry):
            pltpu.sync_copy(dst_h.at[0, pl.ds((win0 + PIPE + t) * CH, CH)],
                            idx_t)
            pltpu.sync_copy(val_h.at[pl.ds((win0 + PIPE + t) * CH, CH)],
                            val_t)
            pltpu.sync_copy(val_t, acc.at[idx_t], add=True)
            return carry

        lax.fori_loop(0, nwin - PIPE, tail_body, 0)

        plsc.subcore_barrier()
        pltpu.sync_copy(acc.at[pl.ds(s * SLICE, SLICE)], slice_b)
        pltpu.sync_copy(slice_b, out_h.at[pl.ds(c * NVP + s * SLICE, SLICE)])

    return k(dst, val)


def _tc_update(g2, vat, partial):
    def body(g_ref, vat_ref, p_ref, out_ref):
        w = g_ref[0, 0]
        full = vat_ref[...]
        a = full[0:1, :]
        b = full[1:2, :]
        x = full[2:3, :]
        cbar = p_ref[0:1, :] + p_ref[1:2, :]
        xn = x + w * (b - cbar) / a
        out_ref[...] = jnp.concatenate([a, b, xn], axis=0)

    return pl.pallas_call(
        body,
        out_shape=jax.ShapeDtypeStruct((3, NVP), jnp.float32),
    )(g2, vat, partial)


def kernel(vertex_attr, edgeij_pair, edge_attr, g, batch):
    dst = edgeij_pair
    val = edge_attr.reshape(N_EDGES)
    partial = _sc_scatter(dst, val).reshape(2, NVP)
    vat = jnp.pad(vertex_attr, ((0, NVP - N_NODES), (0, 0)),
                  constant_values=1.0).T
    out = _tc_update(g.reshape(1, 1), vat, partial)
    return out[:, :N_NODES].T


# speedup vs baseline: 1.0383x; 1.0383x over previous
import functools

import jax
import jax.numpy as jnp
from jax import lax
from jax.experimental import pallas as pl
from jax.experimental.pallas import tpu as pltpu
from jax.experimental.pallas import tpu_sc as plsc

N_NODES = 100000
N_EDGES = 6400000
NVP = 100352
VPW = NVP // 32
FPW = VPW * 3
NW = 32
CH = 2048
NWIN = N_EDGES // CH
WIN_Q = NWIN // NW
WIN_R = NWIN % NW
PIPE = 96
W4 = 8192
NSTEP = PIPE * CH // W4
SLICE = NVP // 16


def _sc_scatter(dst, val):
    mesh = plsc.VectorSubcoreMesh(core_axis_name="c", subcore_axis_name="s")

    @functools.partial(
        pl.kernel,
        out_type=jax.ShapeDtypeStruct((2 * NVP,), jnp.float32),
        mesh=mesh,
        scratch_types=[
            pltpu.VMEM_SHARED((NVP,), jnp.float32),
            pltpu.VMEM((W4,), jnp.int32),
            pltpu.VMEM((W4,), jnp.float32),
            pltpu.VMEM((W4,), jnp.int32),
            pltpu.VMEM((W4,), jnp.float32),
            pltpu.VMEM((CH,), jnp.int32),
            pltpu.VMEM((CH,), jnp.float32),
            pltpu.VMEM((SLICE,), jnp.float32),
            pltpu.SemaphoreType.DMA,
            pltpu.SemaphoreType.DMA,
        ],
    )
    def k(dst_h, val_h, out_h, acc, idx_b0, val_b0, idx_b1, val_b1,
          idx_t, val_t, slice_b, sem0, sem1):
        c = lax.axis_index("c")
        s = lax.axis_index("s")
        w = c * 16 + s

        zero16 = jnp.zeros((16,), jnp.float32)

        def zero_body(i, carry):
            slice_b[pl.ds(i * 16, 16)] = zero16
            return carry

        lax.fori_loop(0, SLICE // 16, zero_body, 0)
        pltpu.sync_copy(slice_b, acc.at[pl.ds(s * SLICE, SLICE)])
        plsc.subcore_barrier()

        win0 = w * WIN_Q + jnp.minimum(w, WIN_R)
        nwin = WIN_Q + jnp.where(w < WIN_R, 1, 0)
        bufs = ((idx_b0, val_b0, sem0), (idx_b1, val_b1, sem1))

        def start_load(step, b):
            idx_b, val_b, sem = bufs[b]
            e = win0 * CH + step * W4
            pltpu.async_copy(dst_h.at[0, pl.ds(e, W4)], idx_b, sem)
            pltpu.async_copy(val_h.at[pl.ds(e, W4)], val_b, sem)

        def wait_load(b):
            idx_b, val_b, sem = bufs[b]
            pltpu.make_async_copy(dst_h.at[0, pl.ds(0, W4)], idx_b, sem).wait()
            pltpu.make_async_copy(val_h.at[pl.ds(0, W4)], val_b, sem).wait()

        def scatter(b):
            idx_b, val_b, _ = bufs[b]
            pltpu.sync_copy(val_b, acc.at[idx_b], add=True)

        start_load(0, 0)
        start_load(1, 1)

        def pair_body(j, carry):
            wait_load(0)
            scatter(0)
            start_load(2 * j + 2, 0)
            wait_load(1)
            scatter(1)
            start_load(2 * j + 3, 1)
            return carry

        lax.fori_loop(0, NSTEP // 2 - 1, pair_body, 0)
        wait_load(0)
        scatter(0)
        wait_load(1)
        scatter(1)

        def tail_body(t, carry):
            pltpu.sync_copy(dst_h.at[0, pl.ds((win0 + PIPE + t) * CH, CH)],
                            idx_t)
            pltpu.sync_copy(val_h.at[pl.ds((win0 + PIPE + t) * CH, CH)],
                            val_t)
            pltpu.sync_copy(val_t, acc.at[idx_t], add=True)
            return carry

        lax.fori_loop(0, nwin - PIPE, tail_body, 0)

        plsc.subcore_barrier()
        pltpu.sync_copy(acc.at[pl.ds(s * SLICE, SLICE)], slice_b)
        pltpu.sync_copy(slice_b, out_h.at[pl.ds(c * NVP + s * SLICE, SLICE)])

    return k(dst, val)


def _tc_update(g2, vat, partial):
    def body(g_ref, vat_ref, p_ref, out_ref):
        w = g_ref[0, 0]
        full = vat_ref[...]
        a = full[0:1, :]
        b = full[1:2, :]
        x = full[2:3, :]
        cbar = (p_ref[0:1, pl.ds(0, N_NODES)]
                + p_ref[1:2, pl.ds(0, N_NODES)])
        xn = x + w * (b - cbar) / a
        out_ref[...] = jnp.concatenate([a, b, xn], axis=0)

    return pl.pallas_call(
        body,
        out_shape=jax.ShapeDtypeStruct((3, N_NODES), jnp.float32),
    )(g2, vat, partial)


def kernel(vertex_attr, edgeij_pair, edge_attr, g, batch):
    dst = edgeij_pair
    val = edge_attr.reshape(N_EDGES)
    partial = _sc_scatter(dst, val).reshape(2, NVP)
    vat = vertex_attr.T
    out = _tc_update(g.reshape(1, 1), vat, partial)
    return out.T
